# baseline (speedup 1.0000x reference)
import jax
import jax.numpy as jnp
from jax import lax
from jax.experimental import pallas as pl
from jax.experimental.pallas import tpu as pltpu
from jax.experimental.pallas import tpu_sc as plsc

_NUM_FIELDS = 26
_FIELD_W = 38462
_BATCH = 16384
_LANES = 16
_NC, _NS = 2, 16
_NW = _NC * _NS
_ROWS = _BATCH // _NW
_FLAT = _ROWS * _NUM_FIELDS
_CH = 4
_CROWS = _ROWS // _CH
_CFLAT = _CROWS * _NUM_FIELDS
_WRAP = _NUM_FIELDS * _FIELD_W
_STEP = _LANES * _FIELD_W


def _body(x_hbm, w_hbm, bias_hbm, out_hbm,
          xv, idx_a, idx_b, vals_a, vals_b, out_v, bias_v, sem_a, sem_b):
    wid = lax.axis_index("s") * _NC + lax.axis_index("c")
    fbase = wid * _FLAT

    pltpu.sync_copy(x_hbm.at[pl.ds(fbase, _FLAT)], xv)
    pltpu.sync_copy(bias_hbm, bias_v)
    bvec = plsc.load_gather(bias_v, [jnp.zeros((_LANES,), jnp.int32)])

    idxb = (idx_a, idx_b)
    valsb = (vals_a, vals_b)
    sems = (sem_a, sem_b)
    handles = [None, None]

    def compute_offsets(k, idx_p):
        cbase = k * _CFLAT

        def off_body(j, f):
            o = j * 2 * _LANES
            idx_p[pl.ds(o, _LANES)] = xv[pl.ds(cbase + o, _LANES)] + f
            f2 = f + _STEP
            f2 = f2 - (f2 >= _WRAP).astype(jnp.int32) * _WRAP
            o2 = o + _LANES
            idx_p[pl.ds(o2, _LANES)] = xv[pl.ds(cbase + o2, _LANES)] + f2
            f3 = f2 + _STEP
            return f3 - (f3 >= _WRAP).astype(jnp.int32) * _WRAP

        lax.fori_loop(0, _CFLAT // (2 * _LANES), off_body,
                      lax.iota(jnp.int32, _LANES) * _FIELD_W)

    def reduce_chunk(k, vals_p):
        rlocal = k * _CROWS

        def red_body(c, _):
            j = c * _LANES + lax.iota(jnp.int32, _LANES)
            base_idx = j * _NUM_FIELDS
            acc = bvec
            for f in range(_NUM_FIELDS):
                acc = acc + plsc.load_gather(vals_p, [base_idx + f])
            out_v[pl.ds(rlocal + c * _LANES, _LANES)] = acc
            return 0

        lax.fori_loop(0, _CROWS // _LANES, red_body, 0)

    for k in range(_CH):
        p = k % 2
        compute_offsets(k, idxb[p])
        handles[p] = pltpu.async_copy(w_hbm.at[idxb[p]], valsb[p], sems[p])
        if k >= 1:
            q = 1 - p
            handles[q].wait()
            reduce_chunk(k - 1, valsb[q])

    last = (_CH - 1) % 2
    handles[last].wait()
    reduce_chunk(_CH - 1, valsb[last])

    pltpu.sync_copy(out_v, out_hbm.at[pl.ds(wid * _ROWS, _ROWS)])


def kernel(x, W, bias):
    x_flat = x.reshape(-1)
    w_flat = W.reshape(-1)
    bias1 = bias.reshape(1)

    run = pl.kernel(
        _body,
        out_type=jax.ShapeDtypeStruct((_BATCH,), jnp.float32),
        mesh=plsc.VectorSubcoreMesh(core_axis_name="c", subcore_axis_name="s"),
        compiler_params=pltpu.CompilerParams(needs_layout_passes=False),
        scratch_types=[
            pltpu.VMEM((_FLAT,), jnp.int32),
            pltpu.VMEM((_CFLAT,), jnp.int32),
            pltpu.VMEM((_CFLAT,), jnp.int32),
            pltpu.VMEM((_CFLAT,), jnp.float32),
            pltpu.VMEM((_CFLAT,), jnp.float32),
            pltpu.VMEM((_ROWS,), jnp.float32),
            pltpu.VMEM((1,), jnp.float32),
            pltpu.SemaphoreType.DMA,
            pltpu.SemaphoreType.DMA,
        ],
    )
    out = run(x_flat, w_flat, bias1)
    return out.reshape(_BATCH, 1)


# speedup vs baseline: 1.0061x; 1.0061x over previous
import jax
import jax.numpy as jnp
from jax import lax
from jax.experimental import pallas as pl
from jax.experimental.pallas import tpu as pltpu
from jax.experimental.pallas import tpu_sc as plsc

_NUM_FIELDS = 26
_FIELD_W = 38462
_BATCH = 16384
_LANES = 16
_NC, _NS = 2, 16
_NW = _NC * _NS
_ROWS = _BATCH // _NW
_FLAT = _ROWS * _NUM_FIELDS
_CH = 8
_CROWS = _ROWS // _CH
_CFLAT = _CROWS * _NUM_FIELDS
_WRAP = _NUM_FIELDS * _FIELD_W
_STEP = _LANES * _FIELD_W


def _body(x_hbm, w_hbm, bias_hbm, out_hbm,
          xv, idx_a, idx_b, vals_a, vals_b, out_v, bias_v, sem_a, sem_b):
    wid = lax.axis_index("s") * _NC + lax.axis_index("c")
    fbase = wid * _FLAT

    pltpu.sync_copy(x_hbm.at[pl.ds(fbase, _FLAT)], xv)
    pltpu.sync_copy(bias_hbm, bias_v)
    bvec = plsc.load_gather(bias_v, [jnp.zeros((_LANES,), jnp.int32)])

    idxb = (idx_a, idx_b)
    valsb = (vals_a, vals_b)
    sems = (sem_a, sem_b)
    handles = [None, None]

    def compute_offsets(k, idx_p):
        cbase = k * _CFLAT

        def off_body(j, f):
            o = j * 2 * _LANES
            idx_p[pl.ds(o, _LANES)] = xv[pl.ds(cbase + o, _LANES)] + f
            f2 = f + _STEP
            f2 = f2 - (f2 >= _WRAP).astype(jnp.int32) * _WRAP
            o2 = o + _LANES
            idx_p[pl.ds(o2, _LANES)] = xv[pl.ds(cbase + o2, _LANES)] + f2
            f3 = f2 + _STEP
            return f3 - (f3 >= _WRAP).astype(jnp.int32) * _WRAP

        lax.fori_loop(0, _CFLAT // (2 * _LANES), off_body,
                      lax.iota(jnp.int32, _LANES) * _FIELD_W)

    def reduce_chunk(k, vals_p):
        rlocal = k * _CROWS

        def red_body(c, _):
            j = c * _LANES + lax.iota(jnp.int32, _LANES)
            base_idx = j * _NUM_FIELDS
            acc = bvec
            for f in range(_NUM_FIELDS):
                acc = acc + plsc.load_gather(vals_p, [base_idx + f])
            out_v[pl.ds(rlocal + c * _LANES, _LANES)] = acc
            return 0

        lax.fori_loop(0, _CROWS // _LANES, red_body, 0)

    for k in range(_CH):
        p = k % 2
        compute_offsets(k, idxb[p])
        handles[p] = pltpu.async_copy(w_hbm.at[idxb[p]], valsb[p], sems[p])
        if k >= 1:
            q = 1 - p
            handles[q].wait()
            reduce_chunk(k - 1, valsb[q])

    last = (_CH - 1) % 2
    handles[last].wait()
    reduce_chunk(_CH - 1, valsb[last])

    pltpu.sync_copy(out_v, out_hbm.at[pl.ds(wid * _ROWS, _ROWS)])


def kernel(x, W, bias):
    x_flat = x.reshape(-1)
    w_flat = W.reshape(-1)
    bias1 = bias.reshape(1)

    run = pl.kernel(
        _body,
        out_type=jax.ShapeDtypeStruct((_BATCH,), jnp.float32),
        mesh=plsc.VectorSubcoreMesh(core_axis_name="c", subcore_axis_name="s"),
        compiler_params=pltpu.CompilerParams(needs_layout_passes=False),
        scratch_types=[
            pltpu.VMEM((_FLAT,), jnp.int32),
            pltpu.VMEM((_CFLAT,), jnp.int32),
            pltpu.VMEM((_CFLAT,), jnp.int32),
            pltpu.VMEM((_CFLAT,), jnp.float32),
            pltpu.VMEM((_CFLAT,), jnp.float32),
            pltpu.VMEM((_ROWS,), jnp.float32),
            pltpu.VMEM((1,), jnp.float32),
            pltpu.SemaphoreType.DMA,
            pltpu.SemaphoreType.DMA,
        ],
    )
    out = run(x_flat, w_flat, bias1)
    return out.reshape(_BATCH, 1)
